# baseline (speedup 1.0000x reference)
import functools

import jax
import jax.numpy as jnp
from jax import lax
from jax.experimental import pallas as pl
from jax.experimental.pallas import tpu as pltpu
from jax.experimental.pallas import tpu_sc as plsc

SEQ_LEN = 200
BATCH = 4096
EMB_DIM = 32
NW = 32
CSTRIPE = BATCH // NW
RBLK = 4
NITER = SEQ_LEN // RBLK


def _gather_body(idx_hbm, table_hbm, out_hbm,
                 idx_all, rows0, rows1, trn0, trn1,
                 gsem0, gsem1, osem0, osem1):
    nc = 2
    wid = lax.axis_index("s") * nc + lax.axis_index("c")
    c0 = wid * CSTRIPE

    rows = (rows0, rows1)
    trn = (trn0, trn1)
    gsem = (gsem0, gsem1)
    osem = (osem0, osem1)

    pltpu.sync_copy(idx_hbm.at[:, pl.ds(c0, CSTRIPE)], idx_all)

    def issue_gathers(i, p):
        for j in range(RBLK):
            pltpu.async_copy(
                table_hbm.at[idx_all.at[i * RBLK + j]], rows[p].at[j], gsem[p])

    def wait_gathers(i, p):
        for j in range(RBLK):
            pltpu.make_async_copy(
                table_hbm.at[idx_all.at[i * RBLK + j]], rows[p].at[j],
                gsem[p]).wait()

    lanes = lax.iota(jnp.int32, 16)
    r_lo, q_lo = lanes // 8, lanes % 8
    r_hi, q_hi = (16 + lanes) // 8, (16 + lanes) % 8

    def transpose(p):
        def trn_j(j, _):
            tj = trn[p].at[j]
            rj = rows[p].at[j]
            for b in range(CSTRIPE):
                bf = jnp.full((16,), b, jnp.int32)
                lo = rj[b, pl.ds(0, 16)]
                hi = rj[b, pl.ds(16, 16)]
                plsc.store_scatter(tj, [r_lo, q_lo, bf], lo)
                plsc.store_scatter(tj, [r_hi, q_hi, bf], hi)
            return _
        lax.fori_loop(0, RBLK, trn_j, None)

    def out_slice(i):
        return out_hbm.at[pl.ds(i * RBLK, RBLK), :, wid, :, :]

    def start_write(i, p):
        pltpu.async_copy(
            trn[p].at[:, :, :, pl.ds(0, CSTRIPE)], out_slice(i), osem[p])

    def wait_write(i, p):
        pltpu.make_async_copy(
            trn[p].at[:, :, :, pl.ds(0, CSTRIPE)], out_slice(i), osem[p]).wait()

    issue_gathers(0, 0)
    issue_gathers(1, 1)

    def step(i, p, first, last):
        if not last:
            issue_gathers(i + 2, p)
        wait_gathers(i, p)
        if not first:
            wait_write(i - 2, p)
        transpose(p)
        start_write(i, p)

    def body(i2, _):
        i = i2 * 2

        wait_gathers(i, 0)

        @pl.when(i2 > 0)
        def _w0():
            wait_write(i - 2, 0)

        transpose(0)
        start_write(i, 0)

        @pl.when(i2 < NITER // 2 - 1)
        def _g0():
            issue_gathers(i + 2, 0)

        wait_gathers(i + 1, 1)

        @pl.when(i2 > 0)
        def _w1():
            wait_write(i - 1, 1)

        transpose(1)
        start_write(i + 1, 1)

        @pl.when(i2 < NITER // 2 - 1)
        def _g1():
            issue_gathers(i + 3, 1)

        return _

    lax.fori_loop(0, NITER // 2, body, None)

    wait_write(NITER - 2, 0)
    wait_write(NITER - 1, 1)


VB = 256
NBLK = 3907
KMAX = 124
VMAX = 1000000 - VB


def _repack_body(tab_hbm, lin_hbm, src0, src1, dst0, dst1,
                 isem0, isem1, osem0, osem1):
    nc = 2
    wid = lax.axis_index("s") * nc + lax.axis_index("c")

    src = (src0, src1)
    dst = (dst0, dst1)
    isem = (isem0, isem1)
    osem = (osem0, osem1)

    def v0_of(k):
        return pl.multiple_of(jnp.minimum((k * NW + wid) * VB, VMAX), VB)

    def start_in(k, p):
        pltpu.async_copy(tab_hbm.at[pl.ds(v0_of(k), VB), :], src[p], isem[p])

    def wait_in(k, p):
        pltpu.make_async_copy(
            tab_hbm.at[pl.ds(v0_of(k), VB), :], src[p], isem[p]).wait()

    def r0_of(k):
        return pl.multiple_of(v0_of(k) // 4, VB // 4)

    def start_out(k, p):
        pltpu.async_copy(
            dst[p], lin_hbm.at[pl.ds(r0_of(k), VB // 4), :], osem[p])

    def wait_out(k, p):
        pltpu.make_async_copy(
            dst[p], lin_hbm.at[pl.ds(r0_of(k), VB // 4), :],
            osem[p]).wait()

    def repack(p):
        def grp(g, _):
            for vi in range(16):
                r = g * 4 + vi // 4
                c = (vi % 4) * 32
                for h in range(2):
                    vals = src[p][g * 16 + vi, pl.ds(h * 16, 16)]
                    dst[p][r, pl.ds(c + h * 16, 16)] = vals
            return _
        lax.fori_loop(0, VB // 16, grp, None)

    start_in(0, 0)
    start_in(1, 1)

    def body(k2, _):
        for sub in range(2):
            k = k2 * 2 + sub
            p = sub
            wait_in(k, p)

            @pl.when(k2 > 0)
            def _w():
                wait_out(k - 2, p)

            repack(p)
            start_out(k, p)

            @pl.when(k2 < KMAX // 2 - 1)
            def _g():
                start_in(k + 2, p)
        return _

    lax.fori_loop(0, KMAX // 2, body, None)
    wait_out(KMAX - 2, 0)
    wait_out(KMAX - 1, 1)


@jax.jit
def _repack_table(table):
    mesh = plsc.VectorSubcoreMesh(core_axis_name="c", subcore_axis_name="s")
    fn = pl.kernel(
        _repack_body,
        out_type=jax.ShapeDtypeStruct((250000, 128), jnp.float32),
        mesh=mesh,
        scratch_types=[
            pltpu.VMEM((VB, EMB_DIM), jnp.float32),
            pltpu.VMEM((VB, EMB_DIM), jnp.float32),
            pltpu.VMEM((VB // 4, 128), jnp.float32),
            pltpu.VMEM((VB // 4, 128), jnp.float32),
            pltpu.SemaphoreType.DMA,
            pltpu.SemaphoreType.DMA,
            pltpu.SemaphoreType.DMA,
            pltpu.SemaphoreType.DMA,
        ],
        compiler_params=pltpu.CompilerParams(
            use_tc_tiling_on_sc=True, needs_layout_passes=False),
    )
    return fn(table)


@jax.jit
def _emb_lookup(idx, table):
    mesh = plsc.VectorSubcoreMesh(core_axis_name="c", subcore_axis_name="s")
    fn = pl.kernel(
        _gather_body,
        out_type=jax.ShapeDtypeStruct(
            (SEQ_LEN, EMB_DIM // 8, NW, 8, CSTRIPE), jnp.float32),
        mesh=mesh,
        scratch_types=[
            pltpu.VMEM((SEQ_LEN, CSTRIPE), jnp.int32),
            pltpu.VMEM((RBLK, CSTRIPE, EMB_DIM), jnp.float32),
            pltpu.VMEM((RBLK, CSTRIPE, EMB_DIM), jnp.float32),
            pltpu.VMEM((RBLK, EMB_DIM // 8, 8, CSTRIPE + 1), jnp.float32),
            pltpu.VMEM((RBLK, EMB_DIM // 8, 8, CSTRIPE + 1), jnp.float32),
            pltpu.SemaphoreType.DMA,
            pltpu.SemaphoreType.DMA,
            pltpu.SemaphoreType.DMA,
            pltpu.SemaphoreType.DMA,
        ],
        compiler_params=pltpu.CompilerParams(
            use_tc_tiling_on_sc=False, needs_layout_passes=False),
    )
    return fn(idx, table)


@jax.jit
def _run(inp, table):
    lin = _repack_table(table)
    out5 = _emb_lookup(inp, lin.reshape(1000000, EMB_DIM))
    return out5.transpose(0, 2, 4, 1, 3).reshape(SEQ_LEN, BATCH, EMB_DIM)


def kernel(inp, lengths, table):
    return _run(inp, table)


# speedup vs baseline: 1.0852x; 1.0852x over previous
import functools

import jax
import jax.numpy as jnp
from jax import lax
from jax.experimental import pallas as pl
from jax.experimental.pallas import tpu as pltpu
from jax.experimental.pallas import tpu_sc as plsc

SEQ_LEN = 200
BATCH = 4096
EMB_DIM = 32
NW = 32
CSTRIPE = BATCH // NW
RBLK = 4
NITER = SEQ_LEN // RBLK


def _gather_body(idx_hbm, table_hbm, out_hbm,
                 idx_all, rows0, rows1, trn0, trn1,
                 gsem0, gsem1, osem0, osem1):
    nc = 2
    wid = lax.axis_index("s") * nc + lax.axis_index("c")
    c0 = wid * CSTRIPE

    rows = (rows0, rows1)
    trn = (trn0, trn1)
    gsem = (gsem0, gsem1)
    osem = (osem0, osem1)

    pltpu.sync_copy(idx_hbm.at[:, pl.ds(c0, CSTRIPE)], idx_all)

    def issue_gathers(i, p):
        for j in range(RBLK):
            pltpu.async_copy(
                table_hbm.at[idx_all.at[i * RBLK + j]], rows[p].at[j], gsem[p])

    def wait_gathers(i, p):
        for j in range(RBLK):
            pltpu.make_async_copy(
                table_hbm.at[idx_all.at[i * RBLK + j]], rows[p].at[j],
                gsem[p]).wait()

    lanes = lax.iota(jnp.int32, 16)
    r_lo, q_lo = lanes // 8, lanes % 8
    r_hi, q_hi = (16 + lanes) // 8, (16 + lanes) % 8

    def transpose(p):
        def trn_j(j, _):
            tj = trn[p].at[j]
            rj = rows[p].at[j]
            for b in range(CSTRIPE):
                bf = jnp.full((16,), b, jnp.int32)
                lo = rj[b, pl.ds(0, 16)]
                hi = rj[b, pl.ds(16, 16)]
                plsc.store_scatter(tj, [r_lo, q_lo, bf], lo)
                plsc.store_scatter(tj, [r_hi, q_hi, bf], hi)
            return _
        lax.fori_loop(0, RBLK, trn_j, None)

    def out_slice(i):
        return out_hbm.at[pl.ds(i * RBLK, RBLK), :, wid, :, :]

    def start_write(i, p):
        pltpu.async_copy(
            trn[p].at[:, :, :, pl.ds(0, CSTRIPE)], out_slice(i), osem[p])

    def wait_write(i, p):
        pltpu.make_async_copy(
            trn[p].at[:, :, :, pl.ds(0, CSTRIPE)], out_slice(i), osem[p]).wait()

    issue_gathers(0, 0)
    issue_gathers(1, 1)

    def step(i, p, first, last):
        if not last:
            issue_gathers(i + 2, p)
        wait_gathers(i, p)
        if not first:
            wait_write(i - 2, p)
        transpose(p)
        start_write(i, p)

    def body(i2, _):
        i = i2 * 2

        wait_gathers(i, 0)

        @pl.when(i2 > 0)
        def _w0():
            wait_write(i - 2, 0)

        transpose(0)
        start_write(i, 0)

        @pl.when(i2 < NITER // 2 - 1)
        def _g0():
            issue_gathers(i + 2, 0)

        wait_gathers(i + 1, 1)

        @pl.when(i2 > 0)
        def _w1():
            wait_write(i - 1, 1)

        transpose(1)
        start_write(i + 1, 1)

        @pl.when(i2 < NITER // 2 - 1)
        def _g1():
            issue_gathers(i + 3, 1)

        return _

    lax.fori_loop(0, NITER // 2, body, None)

    wait_write(NITER - 2, 0)
    wait_write(NITER - 1, 1)


@jax.jit
def _emb_lookup(idx, table):
    mesh = plsc.VectorSubcoreMesh(core_axis_name="c", subcore_axis_name="s")
    fn = pl.kernel(
        _gather_body,
        out_type=jax.ShapeDtypeStruct(
            (SEQ_LEN, EMB_DIM // 8, NW, 8, CSTRIPE), jnp.float32),
        mesh=mesh,
        scratch_types=[
            pltpu.VMEM((SEQ_LEN, CSTRIPE), jnp.int32),
            pltpu.VMEM((RBLK, CSTRIPE, EMB_DIM), jnp.float32),
            pltpu.VMEM((RBLK, CSTRIPE, EMB_DIM), jnp.float32),
            pltpu.VMEM((RBLK, EMB_DIM // 8, 8, CSTRIPE + 1), jnp.float32),
            pltpu.VMEM((RBLK, EMB_DIM // 8, 8, CSTRIPE + 1), jnp.float32),
            pltpu.SemaphoreType.DMA,
            pltpu.SemaphoreType.DMA,
            pltpu.SemaphoreType.DMA,
            pltpu.SemaphoreType.DMA,
        ],
        compiler_params=pltpu.CompilerParams(
            use_tc_tiling_on_sc=False, needs_layout_passes=False),
    )
    return fn(idx, table)


def kernel(inp, lengths, table):
    out5 = _emb_lookup(inp, table)
    return out5.transpose(0, 2, 4, 1, 3).reshape(SEQ_LEN, BATCH, EMB_DIM)
